# baseline (speedup 1.0000x reference)
import functools

import jax
import jax.numpy as jnp
from jax import lax
from jax.experimental import pallas as pl
from jax.experimental.pallas import tpu as pltpu
from jax.experimental.pallas import tpu_sc as plsc

EMBED_DIM = 32
BATCH = 16384
NROWS = 1000000
PIECE = 249984
SWEEP_LIMIT = 4 * PIECE
TAIL = NROWS - SWEEP_LIMIT
WIDE = 128
BN = 11904
NSTEPS = PIECE // BN
NUM_CORES = 2
NUM_SUBCORES = 16
NUM_WORKERS = NUM_CORES * NUM_SUBCORES
B_PER_W = BATCH // NUM_WORKERS
CHUNK = 128
N_CHUNKS = B_PER_W // CHUNK
LANE = 16


def _relayout_body(x0_ref, x1_ref, x2_ref, x3_ref, o_ref):
    xcat = jnp.concatenate(
        [x0_ref[...], x1_ref[...], x2_ref[...], x3_ref[...]], axis=0)
    o_ref[...] = xcat.T


def _tc_relayout(t32):
    def make_map(o):
        return lambda i: (0, NSTEPS * o + i)

    return pl.pallas_call(
        _relayout_body,
        grid=(NSTEPS,),
        in_specs=[pl.BlockSpec((32, BN), make_map(o)) for o in range(4)],
        out_specs=pl.BlockSpec((BN, WIDE), lambda i: (i, 0)),
        out_shape=jax.ShapeDtypeStruct((PIECE, WIDE), jnp.float32),
    )(t32, t32, t32, t32)


def _gather_body(idx_hbm, tab_hbm, out_hbm, idx_v, grp_v, rows_v, sem):
    wid = lax.axis_index("s") * NUM_CORES + lax.axis_index("c")
    base = wid * B_PER_W
    pltpu.sync_copy(idx_hbm.at[pl.ds(base, B_PER_W)], idx_v)
    for v in range(B_PER_W // LANE):
        s = v * LANE
        g = idx_v[pl.ds(s, LANE)]
        for _ in range(4):
            g = jnp.where(g >= PIECE, g - PIECE, g)
        grp_v[pl.ds(s, LANE)] = g
    for j in range(N_CHUNKS):
        s = j * CHUNK
        pltpu.async_copy(tab_hbm.at[grp_v.at[pl.ds(s, CHUNK)]],
                         rows_v.at[pl.ds(s, CHUNK)], sem)
    for j in range(N_CHUNKS):
        s = j * CHUNK
        pltpu.make_async_copy(tab_hbm.at[grp_v.at[pl.ds(s, CHUNK)]],
                              rows_v.at[pl.ds(s, CHUNK)], sem).wait()
    pltpu.sync_copy(rows_v, out_hbm.at[pl.ds(base, B_PER_W)])


def _sc_gather(ids, tab128):
    mesh = plsc.VectorSubcoreMesh(core_axis_name="c", subcore_axis_name="s")
    fn = functools.partial(
        pl.kernel,
        mesh=mesh,
        compiler_params=pltpu.CompilerParams(use_tc_tiling_on_sc=True),
        out_type=jax.ShapeDtypeStruct((BATCH, WIDE), jnp.float32),
        scratch_types=[
            pltpu.VMEM((B_PER_W,), jnp.int32),
            pltpu.VMEM((B_PER_W,), jnp.int32),
            pltpu.VMEM((B_PER_W, WIDE), jnp.float32),
            pltpu.SemaphoreType.DMA,
        ],
    )(_gather_body)
    return fn(ids, tab128)


def _mlp_body(u_ref, m_ref, uid_ref, mid_ref, utail_ref, mtail_ref,
              w1a4_ref, w1b4_ref, w1a_ref, w1b_ref, b1_ref, w2_ref, b2_ref,
              w3_ref, b3_ref, out_ref):
    bm = u_ref.shape[0]
    jj = lax.broadcasted_iota(jnp.int32, (bm, WIDE), 1)

    def contrib(rows, ids, tail, w14, w1):
        ids2 = ids.reshape(-1, 1)
        o32 = (jnp.where(ids2 >= PIECE, 32, 0)
               + jnp.where(ids2 >= 2 * PIECE, 32, 0)
               + jnp.where(ids2 >= 3 * PIECE, 32, 0)
               + jnp.where(ids2 >= SWEEP_LIMIT, 32, 0))
        masked = jnp.where((jj >= o32) & (jj < o32 + 32), rows, 0.0)
        h = jnp.dot(masked, w14, preferred_element_type=jnp.float32)
        onehot = (lax.broadcasted_iota(jnp.int32, (bm, TAIL), 1)
                  == ids2 - SWEEP_LIMIT)
        tw = jnp.dot(tail, w1, preferred_element_type=jnp.float32)
        return h + jnp.dot(onehot.astype(jnp.float32), tw,
                           preferred_element_type=jnp.float32)

    h = contrib(u_ref[...], uid_ref[...], utail_ref[...], w1a4_ref[...],
                w1a_ref[...])
    h = h + contrib(m_ref[...], mid_ref[...], mtail_ref[...], w1b4_ref[...],
                    w1b_ref[...])
    h = jnp.maximum(h + b1_ref[...], 0.0)
    h2 = jnp.dot(h, w2_ref[...], preferred_element_type=jnp.float32)
    h2 = jnp.maximum(h2 + b2_ref[...], 0.0)
    out_ref[...] = jnp.sum(h2 * w3_ref[...], axis=1) + b3_ref[0, 0]


def _tc_mlp(u, m, uids, mids, utail, mtail, W1, b1, W2, b2, W3, b3, bm=2048):
    w1a = W1[:EMBED_DIM]
    w1b = W1[EMBED_DIM:]
    w1a4 = jnp.concatenate([w1a] * 4, axis=0)
    w1b4 = jnp.concatenate([w1b] * 4, axis=0)
    b1r = b1.reshape(1, -1)
    b2r = b2.reshape(1, -1)
    w3r = W3.reshape(1, -1)
    b3r = b3.reshape(1, 1)
    grid = (BATCH // bm,)
    return pl.pallas_call(
        _mlp_body,
        grid=grid,
        in_specs=[
            pl.BlockSpec((bm, WIDE), lambda i: (i, 0)),
            pl.BlockSpec((bm, WIDE), lambda i: (i, 0)),
            pl.BlockSpec((bm,), lambda i: (i,)),
            pl.BlockSpec((bm,), lambda i: (i,)),
            pl.BlockSpec(utail.shape, lambda i: (0, 0)),
            pl.BlockSpec(mtail.shape, lambda i: (0, 0)),
            pl.BlockSpec(w1a4.shape, lambda i: (0, 0)),
            pl.BlockSpec(w1b4.shape, lambda i: (0, 0)),
            pl.BlockSpec(w1a.shape, lambda i: (0, 0)),
            pl.BlockSpec(w1b.shape, lambda i: (0, 0)),
            pl.BlockSpec(b1r.shape, lambda i: (0, 0)),
            pl.BlockSpec(W2.shape, lambda i: (0, 0)),
            pl.BlockSpec(b2r.shape, lambda i: (0, 0)),
            pl.BlockSpec(w3r.shape, lambda i: (0, 0)),
            pl.BlockSpec(b3r.shape, lambda i: (0, 0)),
        ],
        out_specs=pl.BlockSpec((bm,), lambda i: (i,)),
        out_shape=jax.ShapeDtypeStruct((BATCH,), jnp.float32),
    )(u, m, uids, mids, utail, mtail, w1a4, w1b4, w1a, w1b, b1r, W2, b2r,
      w3r, b3r)


def kernel(user_ids, movie_ids, user_table, movie_table, W1, b1, W2, b2, W3,
           b3):
    uids = user_ids.astype(jnp.int32)
    mids = movie_ids.astype(jnp.int32)
    utab128 = _tc_relayout(user_table.T)
    u = _sc_gather(uids, utab128)
    mtab128 = _tc_relayout(movie_table.T)
    m = _sc_gather(mids, mtab128)
    utail = user_table[SWEEP_LIMIT:]
    mtail = movie_table[SWEEP_LIMIT:]
    return _tc_mlp(u, m, uids, mids, utail, mtail, W1, b1, W2, b2, W3, b3)


# speedup vs baseline: 1.0169x; 1.0169x over previous
import functools

import jax
import jax.numpy as jnp
from jax import lax
from jax.experimental import pallas as pl
from jax.experimental.pallas import tpu as pltpu
from jax.experimental.pallas import tpu_sc as plsc

EMBED_DIM = 32
BATCH = 16384
NROWS = 1000000
PIECE = 249984
SWEEP_LIMIT = 4 * PIECE
TAIL = NROWS - SWEEP_LIMIT
WIDE = 128
BN = 11904
NSTEPS = PIECE // BN
NUM_CORES = 2
NUM_SUBCORES = 16
NUM_WORKERS = NUM_CORES * NUM_SUBCORES
B_PER_W = BATCH // NUM_WORKERS
CHUNK = 128
N_CHUNKS = B_PER_W // CHUNK
LANE = 16


def _relayout_body(x0_ref, x1_ref, x2_ref, x3_ref, o_ref):
    xcat = jnp.concatenate(
        [x0_ref[...], x1_ref[...], x2_ref[...], x3_ref[...]], axis=0)
    o_ref[...] = xcat.T


def _tc_relayout(t32):
    def make_map(o):
        return lambda i: (0, NSTEPS * o + i)

    return pl.pallas_call(
        _relayout_body,
        grid=(NSTEPS,),
        in_specs=[pl.BlockSpec((32, BN), make_map(o)) for o in range(4)],
        out_specs=pl.BlockSpec((BN, WIDE), lambda i: (i, 0)),
        out_shape=jax.ShapeDtypeStruct((PIECE, WIDE), jnp.float32),
    )(t32, t32, t32, t32)


def _gather_body(idx_hbm, tab_hbm, out_hbm, idx_v, grp_v, rows_v, sem):
    wid = lax.axis_index("s") * NUM_CORES + lax.axis_index("c")
    base = wid * B_PER_W
    pltpu.sync_copy(idx_hbm.at[pl.ds(base, B_PER_W)], idx_v)
    for v in range(B_PER_W // LANE):
        s = v * LANE
        g = idx_v[pl.ds(s, LANE)]
        for _ in range(4):
            g = jnp.where(g >= PIECE, g - PIECE, g)
        grp_v[pl.ds(s, LANE)] = g
    for j in range(N_CHUNKS):
        s = j * CHUNK
        pltpu.async_copy(tab_hbm.at[grp_v.at[pl.ds(s, CHUNK)]],
                         rows_v.at[pl.ds(s, CHUNK)], sem)
    for j in range(N_CHUNKS):
        s = j * CHUNK
        pltpu.make_async_copy(tab_hbm.at[grp_v.at[pl.ds(s, CHUNK)]],
                              rows_v.at[pl.ds(s, CHUNK)], sem).wait()
    pltpu.sync_copy(rows_v, out_hbm.at[pl.ds(base, B_PER_W)])


def _sc_gather(ids, tab128):
    mesh = plsc.VectorSubcoreMesh(core_axis_name="c", subcore_axis_name="s")
    fn = functools.partial(
        pl.kernel,
        mesh=mesh,
        compiler_params=pltpu.CompilerParams(use_tc_tiling_on_sc=True),
        out_type=jax.ShapeDtypeStruct((BATCH, WIDE), jnp.float32),
        scratch_types=[
            pltpu.VMEM((B_PER_W,), jnp.int32),
            pltpu.VMEM((B_PER_W,), jnp.int32),
            pltpu.VMEM((B_PER_W, WIDE), jnp.float32),
            pltpu.SemaphoreType.DMA,
        ],
    )(_gather_body)
    return fn(ids, tab128)


def _mlp_body(u_ref, m_ref, uid_ref, mid_ref, utail_ref, mtail_ref,
              w1a_ref, w1b_ref, b1_ref, w2_ref, b2_ref, w3_ref, b3_ref,
              out_ref):
    def extract(rows, ids, tail):
        ids2 = ids.reshape(-1, 1)
        x = jnp.where(ids2 < PIECE, rows[:, 0:32], 0.0)
        x = x + jnp.where((ids2 >= PIECE) & (ids2 < 2 * PIECE),
                          rows[:, 32:64], 0.0)
        x = x + jnp.where((ids2 >= 2 * PIECE) & (ids2 < 3 * PIECE),
                          rows[:, 64:96], 0.0)
        x = x + jnp.where((ids2 >= 3 * PIECE) & (ids2 < SWEEP_LIMIT),
                          rows[:, 96:128], 0.0)
        onehot = (lax.broadcasted_iota(jnp.int32, (ids.shape[0], TAIL), 1)
                  == ids2 - SWEEP_LIMIT)
        tfix = jnp.dot(onehot.astype(jnp.float32), tail,
                       preferred_element_type=jnp.float32)
        return jnp.where(ids2 >= SWEEP_LIMIT, tfix, x)

    u = extract(u_ref[...], uid_ref[...], utail_ref[...])
    m = extract(m_ref[...], mid_ref[...], mtail_ref[...])
    h = jnp.dot(u, w1a_ref[...], preferred_element_type=jnp.float32)
    h = h + jnp.dot(m, w1b_ref[...], preferred_element_type=jnp.float32)
    h = jnp.maximum(h + b1_ref[...], 0.0)
    h2 = jnp.dot(h, w2_ref[...], preferred_element_type=jnp.float32)
    h2 = jnp.maximum(h2 + b2_ref[...], 0.0)
    out_ref[...] = jnp.sum(h2 * w3_ref[...], axis=1) + b3_ref[0, 0]


def _tc_mlp(u, m, uids, mids, utail, mtail, W1, b1, W2, b2, W3, b3, bm=2048):
    w1a = W1[:EMBED_DIM]
    w1b = W1[EMBED_DIM:]
    b1r = b1.reshape(1, -1)
    b2r = b2.reshape(1, -1)
    w3r = W3.reshape(1, -1)
    b3r = b3.reshape(1, 1)
    grid = (BATCH // bm,)
    return pl.pallas_call(
        _mlp_body,
        grid=grid,
        in_specs=[
            pl.BlockSpec((bm, WIDE), lambda i: (i, 0)),
            pl.BlockSpec((bm, WIDE), lambda i: (i, 0)),
            pl.BlockSpec((bm,), lambda i: (i,)),
            pl.BlockSpec((bm,), lambda i: (i,)),
            pl.BlockSpec(utail.shape, lambda i: (0, 0)),
            pl.BlockSpec(mtail.shape, lambda i: (0, 0)),
            pl.BlockSpec(w1a.shape, lambda i: (0, 0)),
            pl.BlockSpec(w1b.shape, lambda i: (0, 0)),
            pl.BlockSpec(b1r.shape, lambda i: (0, 0)),
            pl.BlockSpec(W2.shape, lambda i: (0, 0)),
            pl.BlockSpec(b2r.shape, lambda i: (0, 0)),
            pl.BlockSpec(w3r.shape, lambda i: (0, 0)),
            pl.BlockSpec(b3r.shape, lambda i: (0, 0)),
        ],
        out_specs=pl.BlockSpec((bm,), lambda i: (i,)),
        out_shape=jax.ShapeDtypeStruct((BATCH,), jnp.float32),
    )(u, m, uids, mids, utail, mtail, w1a, w1b, b1r, W2, b2r, w3r, b3r)


def kernel(user_ids, movie_ids, user_table, movie_table, W1, b1, W2, b2, W3,
           b3):
    uids = user_ids.astype(jnp.int32)
    mids = movie_ids.astype(jnp.int32)
    utab128 = _tc_relayout(user_table.T)
    u = _sc_gather(uids, utab128)
    mtab128 = _tc_relayout(movie_table.T)
    m = _sc_gather(mids, mtab128)
    utail = user_table[SWEEP_LIMIT:]
    mtail = movie_table[SWEEP_LIMIT:]
    return _tc_mlp(u, m, uids, mids, utail, mtail, W1, b1, W2, b2, W3, b3)
